# baseline (speedup 1.0000x reference)
import functools

import jax
import jax.numpy as jnp
from jax import lax
from jax.experimental import pallas as pl
from jax.experimental.pallas import tpu as pltpu
from jax.experimental.pallas import tpu_sc as plsc

H = 4
NEG_SLOPE = 0.2
EPS = 1e-5
NUM_SC = 2
NUM_TILES = 16
LANES = 16


def _lrelu(v):
    return jnp.maximum(v, NEG_SLOPE * v)


def _tc1_body(nblk, x_ref, w_ref, av_ref, htab_ref, aa_ref, ct_ref):
    r = pl.program_id(0)
    c = pl.program_id(1)
    h = jnp.dot(x_ref[...], w_ref[...], preferred_element_type=jnp.float32,
                precision=jax.lax.Precision.HIGHEST)
    htab_ref[0] = h
    blk = h.shape[0]
    dh = h.shape[1] // H
    ts = (h * av_ref[0][None, :]).reshape(blk, H, dh).sum(-1)
    td = (h * av_ref[1][None, :]).reshape(blk, H, dh).sum(-1)
    part = jnp.concatenate([ts, td], axis=1)

    @pl.when(c == 0)
    def _():
        aa_ref[...] = part

    @pl.when(c == 1)
    def _():
        aa = aa_ref[...] + part
        aa_ref[...] = aa
        m8 = jnp.max(aa, axis=0, keepdims=True)
        mpad = jnp.concatenate([m8, jnp.full((1, 8), -jnp.inf, jnp.float32)],
                               axis=1)

        @pl.when(r == 0)
        def _():
            ct_ref[...] = mpad

        @pl.when(r > 0)
        def _():
            ct_ref[...] = jnp.maximum(ct_ref[...], mpad)

        @pl.when(r == nblk - 1)
        def _():
            m = ct_ref[...]
            cvals = _lrelu(m[:, 0:4] + m[:, 4:8])
            ct_ref[...] = jnp.concatenate(
                [cvals, jnp.zeros((1, 12), jnp.float32)], axis=1)


def _tc1(x, w_perm, av2, nblk, blk):
    n, d = x.shape
    dcols = w_perm.shape[1]
    return pl.pallas_call(
        functools.partial(_tc1_body, nblk),
        grid=(nblk, 2),
        in_specs=[
            pl.BlockSpec((blk, d), lambda r, c: (r, 0)),
            pl.BlockSpec((d, dcols // 2), lambda r, c: (0, c)),
            pl.BlockSpec((2, dcols // 2), lambda r, c: (0, c)),
        ],
        out_specs=[
            pl.BlockSpec((1, blk, dcols // 2), lambda r, c: (c, r, 0)),
            pl.BlockSpec((blk, 2 * H), lambda r, c: (r, 0)),
            pl.BlockSpec((1, 16), lambda r, c: (0, 0)),
        ],
        out_shape=[
            jax.ShapeDtypeStruct((2, n, dcols // 2), jnp.float32),
            jax.ShapeDtypeStruct((n, 2 * H), jnp.float32),
            jax.ShapeDtypeStruct((1, 16), jnp.float32),
        ],
    )(x, w_perm, av2)


BE = 48


def _sc_body(n, e, dh, src_h, dst_h, as_h, ad_h, ct_h, htab_h, zacc_h, zden_h,
             out_h, sp_as, sp_ad, sp_den, sp_acc, v_src, v_dst, v_row, g_idx,
             s_idx, b_log, b_h, b_msg, b_ct, sem_i, sem_g, sem_h, sem_s):
    c = lax.axis_index("c")
    s = lax.axis_index("s")
    OAS, OAD, OEX, ODEN, OATT = (i * H * 128 for i in range(5))

    @pl.when(s == 0)
    def _():
        pltpu.sync_copy(as_h, sp_as)
        pltpu.sync_copy(ad_h, sp_ad)
        pltpu.sync_copy(zden_h, sp_den)
        pltpu.sync_copy(zacc_h, sp_acc)

    pltpu.sync_copy(ct_h.at[0], b_ct)
    plsc.subcore_barrier()

    per_tile = e // NUM_TILES
    n_full = per_tile // BE
    n_rem = (per_tile % BE) // LANES
    base = s * per_tile
    coff = c * n

    def load_chunk(off, be, with_rows):
        c1 = pltpu.async_copy(src_h.at[pl.ds(off, be)],
                              v_src.at[pl.ds(0, be)], sem_i)
        c2 = pltpu.async_copy(dst_h.at[pl.ds(off, be)],
                              v_dst.at[pl.ds(0, be)], sem_i)
        c1.wait()
        c2.wait()
        if with_rows:
            for j in range(be // LANES):
                sl = pl.ds(j * LANES, LANES)
                v_row[sl] = v_src[sl] + coff
            if be > LANES:
                h1 = be - LANES
                hcp = (pltpu.async_copy(htab_h.at[v_row.at[pl.ds(0, h1)]],
                                        b_h.at[pl.ds(0, h1)], sem_h),
                       pltpu.async_copy(htab_h.at[v_row.at[pl.ds(h1, LANES)]],
                                        b_h.at[pl.ds(h1, LANES)], sem_h))
            else:
                hcp = (pltpu.async_copy(htab_h.at[v_row.at[pl.ds(0, be)]],
                                        b_h.at[pl.ds(0, be)], sem_h), None)
        else:
            hcp = None
        for j in range(be // LANES):
            sl = pl.ds(j * LANES, LANES)
            sv = v_src[sl] * H
            dv = v_dst[sl] * H
            for hh in range(H):
                g_idx[pl.ds(hh * 128 + j * LANES, LANES)] = sv + hh
                s_idx[0, hh, sl] = dv + hh
            if with_rows:
                s_idx[0, H, sl] = v_dst[sl]
        return hcp

    def gather_logits(be, with_den):
        cps = []
        for hh in range(H):
            cps.append(pltpu.async_copy(
                sp_as.at[g_idx.at[pl.ds(hh * 128, be)]],
                b_log.at[pl.ds(OAS + hh * 128, be)], sem_g))
            cps.append(pltpu.async_copy(
                sp_ad.at[s_idx.at[0, hh, pl.ds(0, be)]],
                b_log.at[pl.ds(OAD + hh * 128, be)], sem_g))
            if with_den:
                cps.append(pltpu.async_copy(
                    sp_den.at[s_idx.at[0, hh, pl.ds(0, be)]],
                    b_log.at[pl.ds(ODEN + hh * 128, be)], sem_g))
        for cp in cps:
            cp.wait()

    def compute_ex(be):
        cv16 = b_ct[...]
        for hh in range(H):
            cv = jnp.full((LANES,), cv16[hh], jnp.float32)
            for j in range(be // LANES):
                al = (b_log[pl.ds(OAS + hh * 128 + j * LANES, LANES)] +
                      b_log[pl.ds(OAD + hh * 128 + j * LANES, LANES)])
                b_log[pl.ds(OEX + hh * 128 + j * LANES, LANES)] = (
                    jnp.exp(_lrelu(al) - cv))

    def phase1_chunk(off, be):
        load_chunk(off, be, False)
        gather_logits(be, False)
        compute_ex(be)
        cps = []
        for hh in range(H):
            cps.append(pltpu.async_copy(
                b_log.at[pl.ds(OEX + hh * 128, be)],
                sp_den.at[s_idx.at[0, hh, pl.ds(0, be)]], sem_i, add=True))
        for cp in cps:
            cp.wait()

    def phase2_chunk(off, be):
        hcp = load_chunk(off, be, True)
        gather_logits(be, True)
        compute_ex(be)
        inv_h = 1.0 / H
        for hh in range(H):
            for j in range(be // LANES):
                ex = b_log[pl.ds(OEX + hh * 128 + j * LANES, LANES)]
                dn = b_log[pl.ds(ODEN + hh * 128 + j * LANES, LANES)]
                b_log[pl.ds(OATT + hh * 128 + j * LANES, LANES)] = (
                    ex / dn * inv_h)
        hcp1, hcp2 = hcp

        def edge_group(jg, carry):
            at = [b_log[pl.ds(OATT + hh * 128 + jg * LANES, LANES)]
                  for hh in range(H)]
            for k in range(LANES):
                ei = jg * LANES + k
                avs = [jnp.full((LANES,), at[hh][k], jnp.float32)
                       for hh in range(H)]
                for j in range(dh // LANES):
                    m = avs[0] * b_h[ei, pl.ds(j * LANES, LANES)]
                    for hh in range(1, H):
                        m = m + avs[hh] * b_h[ei, pl.ds(hh * dh + j * LANES,
                                                        LANES)]
                    b_msg[ei, pl.ds(j * LANES, LANES)] = m
            return carry

        hcp1.wait()
        if hcp2 is None:
            lax.fori_loop(0, be // LANES, edge_group, 0)
        else:
            lax.fori_loop(0, be // LANES - 1, edge_group, 0)
            hcp2.wait()
            lax.fori_loop(be // LANES - 1, be // LANES, edge_group, 0)
        pltpu.async_copy(b_msg.at[pl.ds(0, be)],
                         sp_acc.at[s_idx.at[0, H, pl.ds(0, be)]], sem_i,
                         add=True).wait()

    def fire_den_scatter():
        for hh in range(H):
            pltpu.async_copy(b_log.at[pl.ds(OEX + hh * 128, BE)],
                             sp_den.at[s_idx.at[0, hh, pl.ds(0, BE)]],
                             sem_s, add=True)

    def wait_den_scatter():
        for hh in range(H):
            pltpu.make_async_copy(
                b_log.at[pl.ds(OEX + hh * 128, BE)],
                sp_den.at[s_idx.at[0, hh, pl.ds(0, BE)]], sem_s).wait()

    if n_full > 0:
        load_chunk(base, BE, False)
        gather_logits(BE, False)
        compute_ex(BE)
        fire_den_scatter()

        def p1(k, carry):
            load_chunk(base + k * BE, BE, False)
            gather_logits(BE, False)
            wait_den_scatter()
            compute_ex(BE)
            fire_den_scatter()
            return carry

        lax.fori_loop(1, n_full, p1, 0)
        wait_den_scatter()
    for k in range(n_rem):
        phase1_chunk(base + n_full * BE + k * LANES, LANES)

    plsc.subcore_barrier()

    def p2(k, carry):
        phase2_chunk(base + k * BE, BE)
        return carry

    lax.fori_loop(0, n_full, p2, 0)
    for k in range(n_rem):
        phase2_chunk(base + n_full * BE + k * LANES, LANES)

    plsc.subcore_barrier()

    rows0 = ((n // NUM_TILES) // 8) * 8
    rlast = n - (NUM_TILES - 1) * rows0

    @pl.when(s < NUM_TILES - 1)
    def _():
        pltpu.sync_copy(sp_acc.at[pl.ds(s * rows0, rows0)],
                        out_h.at[c, pl.ds(s * rows0, rows0)])

    @pl.when(s == NUM_TILES - 1)
    def _():
        pltpu.sync_copy(sp_acc.at[pl.ds((NUM_TILES - 1) * rows0, rlast)],
                        out_h.at[c, pl.ds((NUM_TILES - 1) * rows0, rlast)])


def _sc_aggregate(src, dst, asrc_f, adst_f, ct, htab2, zacc, zden):
    e = src.shape[0]
    n = zacc.shape[0]
    dh = zacc.shape[1]
    dhalf = htab2.shape[1]
    mesh = plsc.VectorSubcoreMesh(core_axis_name="c", subcore_axis_name="s",
                                  num_cores=NUM_SC, num_subcores=NUM_TILES)
    kern = pl.kernel(
        functools.partial(_sc_body, n, e, dh),
        out_type=jax.ShapeDtypeStruct((NUM_SC, n, dh), jnp.float32),
        mesh=mesh,
        scratch_types=[
            pltpu.VMEM_SHARED((n * H,), jnp.float32),
            pltpu.VMEM_SHARED((n * H,), jnp.float32),
            pltpu.VMEM_SHARED((n * H,), jnp.float32),
            pltpu.VMEM_SHARED((n, dh), jnp.float32),
            pltpu.VMEM((128,), jnp.int32),
            pltpu.VMEM((128,), jnp.int32),
            pltpu.VMEM((128,), jnp.int32),
            pltpu.VMEM((H * 128,), jnp.int32),
            pltpu.VMEM((1, 8, 128), jnp.int32),
            pltpu.VMEM((5 * H * 128,), jnp.float32),
            pltpu.VMEM((BE, dhalf), jnp.float32),
            pltpu.VMEM((BE, dh), jnp.float32),
            pltpu.VMEM((16,), jnp.float32),
            pltpu.SemaphoreType.DMA,
            pltpu.SemaphoreType.DMA,
            pltpu.SemaphoreType.DMA,
            pltpu.SemaphoreType.DMA,
        ],
    )
    return kern(src, dst, asrc_f, adst_f, ct, htab2, zacc, zden)


def _tc2_body(o2_ref, s2_ref):
    r = pl.program_id(0)
    o = jnp.concatenate([o2_ref[0], o2_ref[1]], axis=1)
    blk = jnp.stack([o.sum(0), (o * o).sum(0)])

    @pl.when(r == 0)
    def _():
        s2_ref[...] = blk

    @pl.when(r > 0)
    def _():
        s2_ref[...] = s2_ref[...] + blk


def _tc2(o2, nblk, blk):
    n = o2.shape[1]
    dh = o2.shape[2]
    return pl.pallas_call(
        _tc2_body,
        grid=(nblk,),
        in_specs=[pl.BlockSpec((2, blk, dh), lambda r: (0, r, 0))],
        out_specs=pl.BlockSpec((2, 2 * dh), lambda r: (0, 0)),
        out_shape=jax.ShapeDtypeStruct((2, 2 * dh), jnp.float32),
    )(o2)


def _tc3_body(n, o2_ref, s2_ref, g_ref, b_ref, xres_ref, out_ref):
    o = jnp.concatenate([o2_ref[0], o2_ref[1]], axis=1)
    mu = s2_ref[0] / n
    var = s2_ref[1] / n - mu * mu
    y = g_ref[0] * (o - mu[None, :]) * lax.rsqrt(var[None, :] + EPS) + b_ref[0]
    out_ref[...] = jnp.maximum(y, 0.0) + xres_ref[...]


def _tc3(o2, s2, gamma, beta, xres, nblk, blk):
    n = o2.shape[1]
    dh = o2.shape[2]
    d = 2 * dh
    return pl.pallas_call(
        functools.partial(_tc3_body, float(n)),
        grid=(nblk,),
        in_specs=[
            pl.BlockSpec((2, blk, dh), lambda r: (0, r, 0)),
            pl.BlockSpec((2, d), lambda r: (0, 0)),
            pl.BlockSpec((1, d), lambda r: (0, 0)),
            pl.BlockSpec((1, d), lambda r: (0, 0)),
            pl.BlockSpec((blk, d), lambda r: (r, 0)),
        ],
        out_specs=pl.BlockSpec((blk, d), lambda r: (r, 0)),
        out_shape=jax.ShapeDtypeStruct((n, d), jnp.float32),
    )(o2, s2, gamma.reshape(1, d), beta.reshape(1, d), xres)


def _perm_w(w, d, dh):
    return w.reshape(d, H, 2, dh).transpose(0, 2, 1, 3).reshape(d, 2 * H * dh)


def _perm_att(a, dh):
    return a.reshape(H, 2, dh).transpose(1, 0, 2).reshape(2 * H * dh)


def kernel(x, edge_index, W0, att_src0, att_dst0, bias0, gamma0, beta0,
           W1, att_src1, att_dst1, bias1, gamma1, beta1):
    n, d = x.shape
    dh = d // 2
    e = edge_index.shape[1]
    src = edge_index[0]
    dst = edge_index[1]

    blk = n // 10 if n % 10 == 0 else n
    nblk = n // blk
    blk1 = n // 5 if (n % 5 == 0 and (n // 5) % 16 == 0) else n
    nblk1 = n // blk1

    zacc = jnp.zeros((n, dh), jnp.float32)
    zden = jnp.zeros((n * H,), jnp.float32)

    xl = x
    for (w, a_s, a_d, g, b) in (
            (W0, att_src0, att_dst0, gamma0, beta0),
            (W1, att_src1, att_dst1, gamma1, beta1)):
        w_perm = _perm_w(w, d, dh)
        av2 = jnp.stack([_perm_att(a_s, dh), _perm_att(a_d, dh)])
        htab, aa, ct = _tc1(xl, w_perm, av2, nblk1, blk1)
        asrc_f = aa[:, 0:H].reshape(n * H)
        adst_f = aa[:, H:2 * H].reshape(n * H)
        htab2 = htab.reshape(2 * n, H * dh)
        o2 = _sc_aggregate(src, dst, asrc_f, adst_f, ct, htab2, zacc, zden)
        s2 = _tc2(o2, nblk, blk)
        xl = _tc3(o2, s2, g, b, xl, nblk, blk)
    return xl


# speedup vs baseline: 1.0789x; 1.0789x over previous
import functools

import jax
import jax.numpy as jnp
from jax import lax
from jax.experimental import pallas as pl
from jax.experimental.pallas import tpu as pltpu
from jax.experimental.pallas import tpu_sc as plsc

H = 4
NEG_SLOPE = 0.2
EPS = 1e-5
NUM_SC = 2
NUM_TILES = 16
LANES = 16


def _lrelu(v):
    return jnp.maximum(v, NEG_SLOPE * v)


def _tc1_body(nblk, x_ref, w_ref, av_ref, htab_ref, aa_ref, ct_ref):
    r = pl.program_id(0)
    c = pl.program_id(1)
    h = jnp.dot(x_ref[...], w_ref[...], preferred_element_type=jnp.float32,
                precision=jax.lax.Precision.HIGHEST)
    htab_ref[0] = h
    blk = h.shape[0]
    dh = h.shape[1] // H
    ts = (h * av_ref[0][None, :]).reshape(blk, H, dh).sum(-1)
    td = (h * av_ref[1][None, :]).reshape(blk, H, dh).sum(-1)
    part = jnp.concatenate([ts, td], axis=1)

    @pl.when(c == 0)
    def _():
        aa_ref[...] = part

    @pl.when(c == 1)
    def _():
        aa = aa_ref[...] + part
        aa_ref[...] = aa
        m8 = jnp.max(aa, axis=0, keepdims=True)
        mpad = jnp.concatenate([m8, jnp.full((1, 8), -jnp.inf, jnp.float32)],
                               axis=1)

        @pl.when(r == 0)
        def _():
            ct_ref[...] = mpad

        @pl.when(r > 0)
        def _():
            ct_ref[...] = jnp.maximum(ct_ref[...], mpad)

        @pl.when(r == nblk - 1)
        def _():
            m = ct_ref[...]
            cvals = _lrelu(m[:, 0:4] + m[:, 4:8])
            ct_ref[...] = jnp.concatenate(
                [cvals, jnp.zeros((1, 12), jnp.float32)], axis=1)


def _tc1(x, w_perm, av2, nblk, blk):
    n, d = x.shape
    dcols = w_perm.shape[1]
    return pl.pallas_call(
        functools.partial(_tc1_body, nblk),
        grid=(nblk, 2),
        in_specs=[
            pl.BlockSpec((blk, d), lambda r, c: (r, 0)),
            pl.BlockSpec((d, dcols // 2), lambda r, c: (0, c)),
            pl.BlockSpec((2, dcols // 2), lambda r, c: (0, c)),
        ],
        out_specs=[
            pl.BlockSpec((1, blk, dcols // 2), lambda r, c: (c, r, 0)),
            pl.BlockSpec((blk, 2 * H), lambda r, c: (r, 0)),
            pl.BlockSpec((1, 16), lambda r, c: (0, 0)),
        ],
        out_shape=[
            jax.ShapeDtypeStruct((2, n, dcols // 2), jnp.float32),
            jax.ShapeDtypeStruct((n, 2 * H), jnp.float32),
            jax.ShapeDtypeStruct((1, 16), jnp.float32),
        ],
    )(x, w_perm, av2)


BE = 48


def _sc_body(n, e, dh, src_h, dst_h, as_h, ad_h, ct_h, htab_h, zacc_h, zden_h,
             out_h, sp_as, sp_ad, sp_den, sp_acc, v_src, v_dst, v_row, g_idx,
             s_idx, b_log, b_h, b_msg, b_ct, sem_i, sem_g, sem_h, sem_s):
    c = lax.axis_index("c")
    s = lax.axis_index("s")
    OAS, OAD, OEX, ODEN, OATT = (i * H * 128 for i in range(5))

    @pl.when(s == 0)
    def _():
        pltpu.sync_copy(as_h, sp_as)
        pltpu.sync_copy(ad_h, sp_ad)
        pltpu.sync_copy(zden_h, sp_den)
        pltpu.sync_copy(zacc_h, sp_acc)

    pltpu.sync_copy(ct_h.at[0], b_ct)
    plsc.subcore_barrier()

    per_tile = e // NUM_TILES
    n_full = per_tile // BE
    n_rem = (per_tile % BE) // LANES
    base = s * per_tile
    coff = c * n

    def load_chunk(off, be, with_rows):
        c1 = pltpu.async_copy(src_h.at[pl.ds(off, be)],
                              v_src.at[pl.ds(0, be)], sem_i)
        c2 = pltpu.async_copy(dst_h.at[pl.ds(off, be)],
                              v_dst.at[pl.ds(0, be)], sem_i)
        c1.wait()
        c2.wait()
        if with_rows:
            for j in range(be // LANES):
                sl = pl.ds(j * LANES, LANES)
                v_row[sl] = v_src[sl] + coff
            hcp = (pltpu.async_copy(htab_h.at[v_row.at[pl.ds(0, be)]],
                                    b_h.at[pl.ds(0, be)], sem_h), None)
        else:
            hcp = None
        for j in range(be // LANES):
            sl = pl.ds(j * LANES, LANES)
            sv = v_src[sl] * H
            dv = v_dst[sl] * H
            for hh in range(H):
                g_idx[pl.ds(hh * 128 + j * LANES, LANES)] = sv + hh
                s_idx[0, hh, sl] = dv + hh
            if with_rows:
                s_idx[0, H, sl] = v_dst[sl]
        return hcp

    def gather_logits(be, with_den):
        cps = []
        for hh in range(H):
            cps.append(pltpu.async_copy(
                sp_as.at[g_idx.at[pl.ds(hh * 128, be)]],
                b_log.at[pl.ds(OAS + hh * 128, be)], sem_g))
            cps.append(pltpu.async_copy(
                sp_ad.at[s_idx.at[0, hh, pl.ds(0, be)]],
                b_log.at[pl.ds(OAD + hh * 128, be)], sem_g))
            if with_den:
                cps.append(pltpu.async_copy(
                    sp_den.at[s_idx.at[0, hh, pl.ds(0, be)]],
                    b_log.at[pl.ds(ODEN + hh * 128, be)], sem_g))
        for cp in cps:
            cp.wait()

    def compute_ex(be):
        cv16 = b_ct[...]
        for hh in range(H):
            cv = jnp.full((LANES,), cv16[hh], jnp.float32)
            for j in range(be // LANES):
                al = (b_log[pl.ds(OAS + hh * 128 + j * LANES, LANES)] +
                      b_log[pl.ds(OAD + hh * 128 + j * LANES, LANES)])
                b_log[pl.ds(OEX + hh * 128 + j * LANES, LANES)] = (
                    jnp.exp(_lrelu(al) - cv))

    def phase1_chunk(off, be):
        load_chunk(off, be, False)
        gather_logits(be, False)
        compute_ex(be)
        cps = []
        for hh in range(H):
            cps.append(pltpu.async_copy(
                b_log.at[pl.ds(OEX + hh * 128, be)],
                sp_den.at[s_idx.at[0, hh, pl.ds(0, be)]], sem_i, add=True))
        for cp in cps:
            cp.wait()

    def phase2_chunk(off, be):
        hcp = load_chunk(off, be, True)
        gather_logits(be, True)
        compute_ex(be)
        inv_h = 1.0 / H
        for hh in range(H):
            for j in range(be // LANES):
                ex = b_log[pl.ds(OEX + hh * 128 + j * LANES, LANES)]
                dn = b_log[pl.ds(ODEN + hh * 128 + j * LANES, LANES)]
                b_log[pl.ds(OATT + hh * 128 + j * LANES, LANES)] = (
                    ex / dn * inv_h)
        hcp1, hcp2 = hcp

        def edge_group(jg, carry):
            at = [b_log[pl.ds(OATT + hh * 128 + jg * LANES, LANES)]
                  for hh in range(H)]
            for k in range(LANES):
                ei = jg * LANES + k
                avs = [jnp.full((LANES,), at[hh][k], jnp.float32)
                       for hh in range(H)]
                for j in range(dh // LANES):
                    m = avs[0] * b_h[ei, pl.ds(j * LANES, LANES)]
                    for hh in range(1, H):
                        m = m + avs[hh] * b_h[ei, pl.ds(hh * dh + j * LANES,
                                                        LANES)]
                    b_msg[ei, pl.ds(j * LANES, LANES)] = m
            return carry

        hcp1.wait()
        if hcp2 is None:
            lax.fori_loop(0, be // LANES, edge_group, 0)
        else:
            lax.fori_loop(0, be // LANES - 1, edge_group, 0)
            hcp2.wait()
            lax.fori_loop(be // LANES - 1, be // LANES, edge_group, 0)
        pltpu.async_copy(b_msg.at[pl.ds(0, be)],
                         sp_acc.at[s_idx.at[0, H, pl.ds(0, be)]], sem_i,
                         add=True).wait()

    def fire_den_scatter():
        for hh in range(H):
            pltpu.async_copy(b_log.at[pl.ds(OEX + hh * 128, BE)],
                             sp_den.at[s_idx.at[0, hh, pl.ds(0, BE)]],
                             sem_s, add=True)

    def wait_den_scatter():
        for hh in range(H):
            pltpu.make_async_copy(
                b_log.at[pl.ds(OEX + hh * 128, BE)],
                sp_den.at[s_idx.at[0, hh, pl.ds(0, BE)]], sem_s).wait()

    if n_full > 0:
        load_chunk(base, BE, False)
        gather_logits(BE, False)
        compute_ex(BE)
        fire_den_scatter()

        def p1(k, carry):
            load_chunk(base + k * BE, BE, False)
            gather_logits(BE, False)
            wait_den_scatter()
            compute_ex(BE)
            fire_den_scatter()
            return carry

        lax.fori_loop(1, n_full, p1, 0)
        wait_den_scatter()
    for k in range(n_rem):
        phase1_chunk(base + n_full * BE + k * LANES, LANES)

    plsc.subcore_barrier()

    def p2(k, carry):
        phase2_chunk(base + k * BE, BE)
        return carry

    lax.fori_loop(0, n_full, p2, 0)
    for k in range(n_rem):
        phase2_chunk(base + n_full * BE + k * LANES, LANES)

    plsc.subcore_barrier()

    rows0 = ((n // NUM_TILES) // 8) * 8
    rlast = n - (NUM_TILES - 1) * rows0

    @pl.when(s < NUM_TILES - 1)
    def _():
        pltpu.sync_copy(sp_acc.at[pl.ds(s * rows0, rows0)],
                        out_h.at[c, pl.ds(s * rows0, rows0)])

    @pl.when(s == NUM_TILES - 1)
    def _():
        pltpu.sync_copy(sp_acc.at[pl.ds((NUM_TILES - 1) * rows0, rlast)],
                        out_h.at[c, pl.ds((NUM_TILES - 1) * rows0, rlast)])


def _sc_aggregate(src, dst, asrc_f, adst_f, ct, htab2, zacc, zden):
    e = src.shape[0]
    n = zacc.shape[0]
    dh = zacc.shape[1]
    dhalf = htab2.shape[1]
    mesh = plsc.VectorSubcoreMesh(core_axis_name="c", subcore_axis_name="s",
                                  num_cores=NUM_SC, num_subcores=NUM_TILES)
    kern = pl.kernel(
        functools.partial(_sc_body, n, e, dh),
        out_type=jax.ShapeDtypeStruct((NUM_SC, n, dh), jnp.float32),
        mesh=mesh,
        scratch_types=[
            pltpu.VMEM_SHARED((n * H,), jnp.float32),
            pltpu.VMEM_SHARED((n * H,), jnp.float32),
            pltpu.VMEM_SHARED((n * H,), jnp.float32),
            pltpu.VMEM_SHARED((n, dh), jnp.float32),
            pltpu.VMEM((128,), jnp.int32),
            pltpu.VMEM((128,), jnp.int32),
            pltpu.VMEM((128,), jnp.int32),
            pltpu.VMEM((H * 128,), jnp.int32),
            pltpu.VMEM((1, 8, 128), jnp.int32),
            pltpu.VMEM((5 * H * 128,), jnp.float32),
            pltpu.VMEM((BE, dhalf), jnp.float32),
            pltpu.VMEM((BE, dh), jnp.float32),
            pltpu.VMEM((16,), jnp.float32),
            pltpu.SemaphoreType.DMA,
            pltpu.SemaphoreType.DMA,
            pltpu.SemaphoreType.DMA,
            pltpu.SemaphoreType.DMA,
        ],
    )
    return kern(src, dst, asrc_f, adst_f, ct, htab2, zacc, zden)


def _tc2_body(o2_ref, s2_ref):
    r = pl.program_id(0)
    o = jnp.concatenate([o2_ref[0], o2_ref[1]], axis=1)
    blk = jnp.stack([o.sum(0), (o * o).sum(0)])

    @pl.when(r == 0)
    def _():
        s2_ref[...] = blk

    @pl.when(r > 0)
    def _():
        s2_ref[...] = s2_ref[...] + blk


def _tc2(o2, nblk, blk):
    n = o2.shape[1]
    dh = o2.shape[2]
    return pl.pallas_call(
        _tc2_body,
        grid=(nblk,),
        in_specs=[pl.BlockSpec((2, blk, dh), lambda r: (0, r, 0))],
        out_specs=pl.BlockSpec((2, 2 * dh), lambda r: (0, 0)),
        out_shape=jax.ShapeDtypeStruct((2, 2 * dh), jnp.float32),
    )(o2)


def _tc3_body(n, o2_ref, s2_ref, g_ref, b_ref, xres_ref, out_ref):
    o = jnp.concatenate([o2_ref[0], o2_ref[1]], axis=1)
    mu = s2_ref[0] / n
    var = s2_ref[1] / n - mu * mu
    y = g_ref[0] * (o - mu[None, :]) * lax.rsqrt(var[None, :] + EPS) + b_ref[0]
    out_ref[...] = jnp.maximum(y, 0.0) + xres_ref[...]


def _tc3(o2, s2, gamma, beta, xres, nblk, blk):
    n = o2.shape[1]
    dh = o2.shape[2]
    d = 2 * dh
    return pl.pallas_call(
        functools.partial(_tc3_body, float(n)),
        grid=(nblk,),
        in_specs=[
            pl.BlockSpec((2, blk, dh), lambda r: (0, r, 0)),
            pl.BlockSpec((2, d), lambda r: (0, 0)),
            pl.BlockSpec((1, d), lambda r: (0, 0)),
            pl.BlockSpec((1, d), lambda r: (0, 0)),
            pl.BlockSpec((blk, d), lambda r: (r, 0)),
        ],
        out_specs=pl.BlockSpec((blk, d), lambda r: (r, 0)),
        out_shape=jax.ShapeDtypeStruct((n, d), jnp.float32),
    )(o2, s2, gamma.reshape(1, d), beta.reshape(1, d), xres)


def _perm_w(w, d, dh):
    return w.reshape(d, H, 2, dh).transpose(0, 2, 1, 3).reshape(d, 2 * H * dh)


def _perm_att(a, dh):
    return a.reshape(H, 2, dh).transpose(1, 0, 2).reshape(2 * H * dh)


def kernel(x, edge_index, W0, att_src0, att_dst0, bias0, gamma0, beta0,
           W1, att_src1, att_dst1, bias1, gamma1, beta1):
    n, d = x.shape
    dh = d // 2
    e = edge_index.shape[1]
    src = edge_index[0]
    dst = edge_index[1]

    blk = n // 10 if n % 10 == 0 else n
    nblk = n // blk
    blk1 = n // 5 if (n % 5 == 0 and (n // 5) % 16 == 0) else n
    nblk1 = n // blk1

    zacc = jnp.zeros((n, dh), jnp.float32)
    zden = jnp.zeros((n * H,), jnp.float32)

    xl = x
    for (w, a_s, a_d, g, b) in (
            (W0, att_src0, att_dst0, gamma0, beta0),
            (W1, att_src1, att_dst1, gamma1, beta1)):
        w_perm = _perm_w(w, d, dh)
        av2 = jnp.stack([_perm_att(a_s, dh), _perm_att(a_d, dh)])
        htab, aa, ct = _tc1(xl, w_perm, av2, nblk1, blk1)
        asrc_f = aa[:, 0:H].reshape(n * H)
        adst_f = aa[:, H:2 * H].reshape(n * H)
        htab2 = htab.reshape(2 * n, H * dh)
        o2 = _sc_aggregate(src, dst, asrc_f, adst_f, ct, htab2, zacc, zden)
        s2 = _tc2(o2, nblk, blk)
        xl = _tc3(o2, s2, g, b, xl, nblk, blk)
    return xl
